# Initial kernel scaffold; baseline (speedup 1.0000x reference)
import jax
import jax.numpy as jnp
from jax.experimental import pallas as pl


def kernel(f_atoms, f_bonds, edge_index, t_atoms, t_bonds, W_i, W_h, W_o, ffn_W, ffn_b):
    raise NotImplementedError("write your pallas kernel here")


# speedup vs baseline: 2.3534x; 2.3534x over previous
import functools

import jax
import jax.numpy as jnp
from jax import lax
from jax.experimental import pallas as pl
from jax.experimental.pallas import tpu as pltpu
from jax.experimental.pallas import tpu_sc as plsc

_NC = 2
_NS = 16
_NW = _NC * _NS
_CHUNK = 80
_LANES = 16


def _sc_pass(stream, table, src, dst, zeros, write_h):
    n_edges, feat = stream.shape
    n_nodes = table.shape[0]
    per_tile = n_edges // _NW
    chunks_per_tile = per_tile // _CHUNK
    rows_main = (n_nodes // _NS) // 8 * 8
    rows_tail = n_nodes - _NS * rows_main
    n_slices = feat // _LANES

    out_type = [jax.ShapeDtypeStruct((_NC * n_nodes, feat), jnp.float32)]
    if write_h:
        out_type.append(jax.ShapeDtypeStruct((n_edges, feat), jnp.float32))

    def body(stream_hbm, table_hbm, src_hbm, dst_hbm, zeros_hbm, *rest):
        if write_h:
            a_hbm, h_hbm, sbuf, gbuf, sidx, didx, acc = rest
        else:
            a_hbm, sbuf, gbuf, sidx, didx, acc = rest
            h_hbm = None
        cid = lax.axis_index("c")
        sid = lax.axis_index("s")
        wid = cid * _NS + sid

        pltpu.sync_copy(zeros_hbm.at[pl.ds(sid * rows_main, rows_main)],
                        acc.at[pl.ds(sid * rows_main, rows_main)])
        if rows_tail:
            @pl.when(sid == _NS - 1)
            def _tail_zero():
                pltpu.sync_copy(
                    zeros_hbm.at[pl.ds(_NS * rows_main, rows_tail)],
                    acc.at[pl.ds(_NS * rows_main, rows_tail)])
        plsc.subcore_barrier()

        base0 = wid * per_tile

        @pl.loop(0, chunks_per_tile)
        def _chunk(c):
            base = base0 + c * _CHUNK
            pltpu.sync_copy(src_hbm.at[pl.ds(base, _CHUNK)], sidx)
            pltpu.sync_copy(dst_hbm.at[pl.ds(base, _CHUNK)], didx)
            pltpu.sync_copy(stream_hbm.at[pl.ds(base, _CHUNK)], sbuf)
            pltpu.sync_copy(table_hbm.at[sidx], gbuf)

            @pl.loop(0, _CHUNK)
            def _row(r):
                for j in range(n_slices):
                    slc = (pl.ds(r, 1), pl.ds(j * _LANES, _LANES))
                    sbuf.at[*slc][...] = jnp.maximum(
                        sbuf.at[*slc][...] + gbuf.at[*slc][...], 0.0)

            if write_h:
                pltpu.sync_copy(sbuf, h_hbm.at[pl.ds(base, _CHUNK)])
            pltpu.sync_copy(sbuf, acc.at[didx], add=True)

        plsc.subcore_barrier()
        pltpu.sync_copy(
            acc.at[pl.ds(sid * rows_main, rows_main)],
            a_hbm.at[pl.ds(cid * n_nodes + sid * rows_main, rows_main)])
        if rows_tail:
            @pl.when(sid == _NS - 1)
            def _tail_out():
                pltpu.sync_copy(
                    acc.at[pl.ds(_NS * rows_main, rows_tail)],
                    a_hbm.at[pl.ds(cid * n_nodes + _NS * rows_main, rows_tail)])

    kern = pl.kernel(
        body,
        out_type=out_type,
        mesh=plsc.VectorSubcoreMesh(core_axis_name="c", subcore_axis_name="s"),
        scratch_types=[
            pltpu.VMEM((_CHUNK, feat), jnp.float32),
            pltpu.VMEM((_CHUNK, feat), jnp.float32),
            pltpu.VMEM((_CHUNK,), jnp.int32),
            pltpu.VMEM((_CHUNK,), jnp.int32),
            pltpu.VMEM_SHARED((n_nodes, feat), jnp.float32),
        ],
    )
    res = kern(stream, table, src, dst, zeros)
    if write_h:
        return res[0], res[1]
    return res[0], None


def _tc_matmul(x, w, block_rows):
    n, k = x.shape
    _, m = w.shape

    def body(x_ref, w_ref, o_ref):
        o_ref[...] = jnp.dot(x_ref[...], w_ref[...],
                             preferred_element_type=jnp.float32)

    return pl.pallas_call(
        body,
        grid=(n // block_rows,),
        in_specs=[pl.BlockSpec((block_rows, k), lambda i: (i, 0)),
                  pl.BlockSpec((k, m), lambda i: (0, 0))],
        out_specs=pl.BlockSpec((block_rows, m), lambda i: (i, 0)),
        out_shape=jax.ShapeDtypeStruct((n, m), jnp.float32),
    )(x, w)


def _tc_message(a_parts, w_h, n_nodes, block_rows):
    h = w_h.shape[0]
    nb = n_nodes // block_rows

    def body(p0_ref, p1_ref, w_ref, o_ref):
        o_ref[...] = jnp.dot(p0_ref[...] + p1_ref[...], w_ref[...],
                             preferred_element_type=jnp.float32)

    return pl.pallas_call(
        body,
        grid=(nb,),
        in_specs=[pl.BlockSpec((block_rows, h), lambda i: (i, 0)),
                  pl.BlockSpec((block_rows, h), lambda i: (i + nb, 0)),
                  pl.BlockSpec((h, h), lambda i: (0, 0))],
        out_specs=pl.BlockSpec((block_rows, h), lambda i: (i, 0)),
        out_shape=jax.ShapeDtypeStruct((n_nodes, h), jnp.float32),
    )(a_parts, a_parts, w_h)


def _tc_final(f_atoms, a_parts, w_o, ffn_w, ffn_b, block_rows):
    n, d_atom = f_atoms.shape
    h = w_o.shape[1]
    nb = n // block_rows

    def body(fa_ref, p0_ref, p1_ref, wo_ref, fw_ref, fb_ref, v_ref, r_ref):
        agg = p0_ref[...] + p1_ref[...]
        x = jnp.dot(fa_ref[...], wo_ref[0:d_atom, :],
                    preferred_element_type=jnp.float32)
        x = x + jnp.dot(agg, wo_ref[d_atom:, :],
                        preferred_element_type=jnp.float32)
        v = jnp.maximum(x, 0.0)
        v_ref[...] = v
        r_ref[...] = (jnp.dot(v, fw_ref[...], preferred_element_type=jnp.float32)
                      + fb_ref[0, 0])

    return pl.pallas_call(
        body,
        grid=(nb,),
        in_specs=[pl.BlockSpec((block_rows, d_atom), lambda i: (i, 0)),
                  pl.BlockSpec((block_rows, h), lambda i: (i, 0)),
                  pl.BlockSpec((block_rows, h), lambda i: (i + nb, 0)),
                  pl.BlockSpec((d_atom + h, h), lambda i: (0, 0)),
                  pl.BlockSpec((h, 1), lambda i: (0, 0)),
                  pl.BlockSpec((1, 1), lambda i: (0, 0))],
        out_specs=[pl.BlockSpec((block_rows, h), lambda i: (i, 0)),
                   pl.BlockSpec((block_rows, 1), lambda i: (i, 0))],
        out_shape=[jax.ShapeDtypeStruct((n, h), jnp.float32),
                   jax.ShapeDtypeStruct((n, 1), jnp.float32)],
    )(f_atoms, a_parts, a_parts, w_o, ffn_w, ffn_b)


def _tc_matvec_bias(x, w, b, block_rows):
    n, k = x.shape

    def body(x_ref, w_ref, b_ref, o_ref):
        o_ref[...] = (jnp.dot(x_ref[...], w_ref[...],
                              preferred_element_type=jnp.float32) + b_ref[0, 0])

    return pl.pallas_call(
        body,
        grid=(n // block_rows,),
        in_specs=[pl.BlockSpec((block_rows, k), lambda i: (i, 0)),
                  pl.BlockSpec((k, 1), lambda i: (0, 0)),
                  pl.BlockSpec((1, 1), lambda i: (0, 0))],
        out_specs=pl.BlockSpec((block_rows, 1), lambda i: (i, 0)),
        out_shape=jax.ShapeDtypeStruct((n, 1), jnp.float32),
    )(x, w, b)


def kernel(f_atoms, f_bonds, edge_index, t_atoms, t_bonds, W_i, W_h, W_o,
           ffn_W, ffn_b):
    n_nodes, d_atom = f_atoms.shape
    n_edges, d_bond = f_bonds.shape
    hdim = W_h.shape[0]

    src = edge_index[0]
    dst = edge_index[1]
    zeros = jnp.zeros((n_nodes, hdim), jnp.float32)
    ffn_b2 = ffn_b.reshape(1, 1)

    pa = _tc_matmul(f_atoms, W_i[:d_atom], block_rows=2000)
    pb = _tc_matmul(f_bonds, W_i[d_atom:], block_rows=4000)

    a_parts, h0 = _sc_pass(pb, pa, src, dst, zeros, write_h=True)
    m = _tc_message(a_parts, W_h, n_nodes, block_rows=2000)
    a_parts, _ = _sc_pass(h0, m, src, dst, zeros, write_h=False)
    m = _tc_message(a_parts, W_h, n_nodes, block_rows=2000)
    a_parts, _ = _sc_pass(h0, m, src, dst, zeros, write_h=False)
    m = _tc_message(a_parts, W_h, n_nodes, block_rows=2000)
    a_parts, h3 = _sc_pass(h0, m, src, dst, zeros, write_h=True)

    atoms_v, r_atoms = _tc_final(f_atoms, a_parts, W_o, ffn_W, ffn_b2,
                                 block_rows=2000)
    r_bonds = _tc_matvec_bias(h3, ffn_W, ffn_b2, block_rows=8000)

    r_all = jnp.concatenate([r_bonds, r_atoms], axis=0)
    t_all = jnp.concatenate([t_bonds, t_atoms[1:]], axis=0)
    v_all = jnp.concatenate([h3, atoms_v], axis=0)
    return (r_all, t_all, v_all)
